# baseline (speedup 1.0000x reference)
import jax
import jax.numpy as jnp
from jax.experimental import pallas as pl

E = 50000
N_REL = 24
T = 3
TAU_1 = 10.0
TOP_K = 1000
TOP_K_MASK = 20000


def _lstm(seq_tbe, W_ih, W_hh, b_ih, b_hh):
    Bm = seq_tbe.shape[1]
    H = W_hh.shape[1]

    def step(carry, x_t):
        h, c = carry
        g = x_t @ W_ih.T + b_ih + h @ W_hh.T + b_hh
        i, f, gg, o = jnp.split(g, 4, axis=-1)
        i = jax.nn.sigmoid(i)
        f = jax.nn.sigmoid(f)
        gg = jnp.tanh(gg)
        o = jax.nn.sigmoid(o)
        c = f * c + i * gg
        h = o * jnp.tanh(c)
        return (h, c), h

    init = (jnp.zeros((Bm, H), dtype=seq_tbe.dtype), jnp.zeros((Bm, H), dtype=seq_tbe.dtype))
    _, hs = jax.lax.scan(step, init, seq_tbe)
    return hs


def _topk_mask(x, k):
    vals = jax.lax.top_k(x, k)[0]
    thr = vals[:, -1:]
    return jnp.where(x >= thr, x, jnp.zeros_like(x))


def kernel(input_x, input_r, e2triple, triple2e, r2triple,
           emb_table, W_ih, W_hh, b_ih, b_hh, lin_W, lin_b):
    Bm = input_x.shape[0]
    head = e2triple[0]
    rel_inv = e2triple[2] % N_REL
    tail = triple2e[1]
    rel = r2triple[0]

    x_ori = jax.nn.one_hot(input_x, E, dtype=jnp.float32)
    input_emb_ori = emb_table[input_r]
    seq = jnp.stack([input_emb_ori] * (T + 1), axis=1)
    seq = seq.at[:, -1, :].set(jnp.broadcast_to(emb_table[N_REL], (Bm, 128)))
    rnn = _lstm(jnp.transpose(seq, (1, 0, 2)), W_ih, W_hh, b_ih, b_hh)
    rnn = jnp.transpose(rnn, (1, 0, 2))
    w_all = rnn[:, :-1, :] @ lin_W.T + lin_b

    states = [x_ori[:, None, :]]
    for t in range(T + 1):
        a = rnn[:, t]
        bt = rnn[:, :t + 1]
        att = jax.nn.softmax(jnp.einsum('bd,btd->bt', a, bt), axis=-1)
        memory = jnp.stack(states, axis=0)
        inp = jnp.einsum('bt,tbln->bln', att, memory)
        if t < T:
            w = jax.nn.softmax(w_all[:, t, :] / TAU_1, axis=-1)
            x = inp[:, 0, :]
            if t >= 1:
                x = _topk_mask(x, min(TOP_K_MASK, E))
            msg_h = jnp.take(x, head, axis=1) * jnp.take(w, rel, axis=1)
            s_h = jnp.zeros((Bm, E), dtype=x.dtype).at[:, tail].add(msg_h)
            msg_t = jnp.take(x, tail, axis=1) * jnp.take(w, rel_inv, axis=1)
            s_t = jnp.zeros((Bm, E), dtype=x.dtype).at[:, head].add(msg_t)
            s = s_h + s_t
            if t >= 1:
                s = _topk_mask(s, TOP_K)
            s = s[:, None, :]
            s_sum = jnp.sum(s, axis=-1, keepdims=True)
            s = s / jnp.clip(s_sum, 1e-7, None)
        else:
            s = inp
        states.append(s)
    return states[-1].sum(axis=1)


# speedup vs baseline: 32.1300x; 32.1300x over previous
import functools

import jax
import jax.numpy as jnp
from jax import lax
from jax.experimental import pallas as pl
from jax.experimental.pallas import tpu as pltpu
from jax.experimental.pallas import tpu_sc as plsc

E = 50000
E_PAD = 50048
BP = 8
B = 8
N_REL = 24
TAU = 10.0
N_TRIPLES = 800000
NW = 16
W = 1600
M = NW // 2
N_PAD = 2 * 16 * NW * W
ROWS_TC = E_PAD * BP // 128
TOP_K = 1000
TOP_K_MASK = 20000
F32_INF_BITS = 0x7F800000


def _dot(a, b):
    return lax.dot_general(a, b, (((1,), (1,)), ((), ())),
                           precision=lax.Precision.HIGHEST,
                           preferred_element_type=jnp.float32)


def _sigmoid(x):
    return 1.0 / (1.0 + jnp.exp(-x))


def _fold8(v):
    v = v + pltpu.roll(v, 8, 1)
    v = v + pltpu.roll(v, 16, 1)
    v = v + pltpu.roll(v, 32, 1)
    v = v + pltpu.roll(v, 64, 1)
    return v


def _thr_bits(bits, k):
    lo0 = jnp.zeros((1, 128), jnp.int32)
    hi0 = jnp.full((1, 128), F32_INF_BITS, jnp.int32)
    kf = jnp.float32(k)

    def body(_, c):
        lo, hi = c
        mid = lo + ((hi - lo + 1) >> 1)
        pred = (bits >= mid).astype(jnp.float32)
        cnt = _fold8(jnp.sum(pred, axis=0, keepdims=True))
        ge = cnt >= kf
        return jnp.where(ge, mid, lo), jnp.where(ge, hi, mid - 1)

    lo, _ = lax.fori_loop(0, 31, body, (lo0, hi0))
    return lo


def _topk_mask(x, k):
    bits = lax.bitcast_convert_type(x, jnp.int32)
    thr = _thr_bits(bits, k)
    return jnp.where(bits >= thr, x, jnp.zeros_like(x))


def _topk_mask_or_identity(x, k):
    bits = lax.bitcast_convert_type(x, jnp.int32)
    nz = _fold8(jnp.sum((bits >= 1).astype(jnp.float32), axis=0,
                        keepdims=True))
    any_full = jnp.max(nz) >= jnp.float32(k)
    return lax.cond(any_full, lambda: _topk_mask(x, k), lambda: x)


def _k1_body(emb_seq_ref, emb_last_ref, wih_ref, whh_ref, b_ref, linw_ref,
             linb_ref, xb_ref, x0_ref, att_ref, wsm_ref):
    Wih = wih_ref[...]
    Whh = whh_ref[...]
    bias = b_ref[...]
    h = jnp.zeros((B, 128), jnp.float32)
    c = jnp.zeros((B, 128), jnp.float32)
    rnn = []
    for t in range(4):
        x_t = emb_seq_ref[...] if t < 3 else jnp.broadcast_to(
            emb_last_ref[...], (B, 128))
        g = _dot(x_t, Wih) + _dot(h, Whh) + bias
        i = _sigmoid(g[:, 0:128])
        f = _sigmoid(g[:, 128:256])
        gg = jnp.tanh(g[:, 256:384])
        o = _sigmoid(g[:, 384:512])
        c = f * c + i * gg
        h = o * jnp.tanh(c)
        rnn.append(h)

    tau_iota = lax.broadcasted_iota(jnp.int32, (1, 4), 1)
    for t in range(4):
        scores = jnp.concatenate(
            [jnp.sum(rnn[t] * rnn[tau], axis=1, keepdims=True)
             for tau in range(4)], axis=1)
        valid = tau_iota <= t
        z = jnp.where(valid, scores, -1e30)
        m = jnp.max(z, axis=1, keepdims=True)
        e = jnp.exp(z - m) * valid.astype(jnp.float32)
        att_ref[t] = e / jnp.sum(e, axis=1, keepdims=True)

    linW = linw_ref[...]
    linb = linb_ref[...]
    for t in range(3):
        wa = (_dot(rnn[t], linW) + linb) / TAU
        m = jnp.max(wa, axis=1, keepdims=True)
        e = jnp.exp(wa - m)
        wsm_ref[t] = e / jnp.sum(e, axis=1, keepdims=True)

    R = lax.broadcasted_iota(jnp.int32, (ROWS_TC, 128), 0)
    L = lax.broadcasted_iota(jnp.int32, (ROWS_TC, 128), 1)
    ent = R * 16 + (L >> 3)
    x0_ref[...] = (ent == xb_ref[...]).astype(jnp.float32)


def _run_k1(emb_seq, emb_last, W_ih, W_hh, bsum, lin_W, linb, xb):
    return pl.pallas_call(
        _k1_body,
        out_shape=[
            jax.ShapeDtypeStruct((ROWS_TC, 128), jnp.float32),
            jax.ShapeDtypeStruct((4, B, 4), jnp.float32),
            jax.ShapeDtypeStruct((3, B, N_REL), jnp.float32),
        ],
    )(emb_seq, emb_last, W_ih, W_hh, bsum, lin_W, linb, xb)


def _make_step(t):
    def body(*refs):
        p_ref = refs[0]
        coef_ref = refs[1]
        state_refs = refs[2:2 + (t + 1)]
        snew_ref = refs[2 + (t + 1)]
        xnext_ref = refs[3 + (t + 1)]

        s = p_ref[0] + p_ref[1]
        if t >= 1:
            s = _topk_mask(s, TOP_K)
        rs = _fold8(jnp.sum(s, axis=0, keepdims=True))
        denom = jnp.maximum(rs, 1e-7)
        snew = s / denom
        snew_ref[...] = snew

        xn = coef_ref[t + 1] * snew
        for tau in range(t + 1):
            xn = xn + coef_ref[tau] * state_refs[tau][...]
        if t + 1 <= 2:
            xn = _topk_mask_or_identity(xn, TOP_K_MASK)
        xnext_ref[...] = xn

    return body


def _run_step(t, p2, cf, states2d):
    return pl.pallas_call(
        _make_step(t),
        out_shape=[
            jax.ShapeDtypeStruct((ROWS_TC, 128), jnp.float32),
            jax.ShapeDtypeStruct((ROWS_TC, 128), jnp.float32),
        ],
    )(p2, cf, *states2d)


_SC_ROWS = E_PAD // 16


@functools.cache
def _build_sc_propagate():
  mesh = plsc.VectorSubcoreMesh(core_axis_name="c", subcore_axis_name="s")

  @functools.partial(
    pl.kernel,
    out_type=jax.ShapeDtypeStruct((2, E_PAD, BP), jnp.float32),
    mesh=mesh,
    compiler_params=pltpu.CompilerParams(use_tc_tiling_on_sc=False,
                                         needs_layout_passes=False),
    scratch_types=[
        pltpu.VMEM_SHARED((E_PAD, BP), jnp.float32),
        pltpu.VMEM_SHARED((E_PAD, BP), jnp.float32),
        pltpu.VMEM_SHARED((32, BP), jnp.float32),
        pltpu.VMEM((W,), jnp.int32),
        pltpu.VMEM((W,), jnp.int32),
        pltpu.VMEM((W,), jnp.int32),
        pltpu.VMEM((W,), jnp.int32),
        pltpu.VMEM((W,), jnp.int32),
        pltpu.VMEM((W,), jnp.int32),
        pltpu.VMEM((W,), jnp.int32),
        pltpu.VMEM((W,), jnp.int32),
        pltpu.VMEM((W, BP), jnp.float32),
        pltpu.VMEM((W, BP), jnp.float32),
        pltpu.VMEM((W, BP), jnp.float32),
        pltpu.VMEM((W, BP), jnp.float32),
        pltpu.SemaphoreType.DMA,
        pltpu.SemaphoreType.DMA,
        pltpu.SemaphoreType.DMA,
        pltpu.SemaphoreType.DMA,
        pltpu.SemaphoreType.DMA,
        pltpu.SemaphoreType.DMA,
    ],
  )
  def _sc_propagate(x_hbm, w_hbm, z_hbm, h_hbm, t_hbm, r_hbm, ri_hbm, out_hbm,
                    xsp, ssp, wsp, ga0, ga1, wa0, wa1, sa0, sa1, sa2, sa3,
                    xg0, xg1, wg0, wg1, si0, si1, sg0, sg1, ss0, ss1):
    c = lax.axis_index("c")
    s = lax.axis_index("s")
    r0 = s * _SC_ROWS
    pltpu.sync_copy(x_hbm.at[pl.ds(r0, _SC_ROWS)], xsp.at[pl.ds(r0, _SC_ROWS)])
    pltpu.sync_copy(z_hbm.at[pl.ds(r0, _SC_ROWS)], ssp.at[pl.ds(r0, _SC_ROWS)])

    @pl.when(s == 0)
    def _():
        pltpu.sync_copy(w_hbm, wsp)

    plsc.subcore_barrier()

    iota = lax.iota(jnp.int32, 16)
    rbase = iota >> 3
    cidx = iota & 7
    ga = (ga0, ga1)
    wa = (wa0, wa1)
    sa = (sa0, sa1, sa2, sa3)
    xg = (xg0, xg1)
    wg = (wg0, wg1)
    si = (si0, si1)
    sg = (sg0, sg1)
    ss = (ss0, ss1)

    Q = NW // 4

    def run_pass(gsrc, wsrc, ssrc):
        def idx_copies(j, u):
            p = u % 2
            return (pltpu.make_async_copy(gsrc.at[c, s, j], ga[p], si[p]),
                    pltpu.make_async_copy(wsrc.at[c, s, j], wa[p], si[p]),
                    pltpu.make_async_copy(ssrc.at[c, s, j], sa[u], si[p]))

        def idx_start(j, u):
            for d in idx_copies(j, u):
                d.start()

        def idx_wait(j, u):
            for d in idx_copies(j, u):
                d.wait()

        def gath_copies(p):
            return (pltpu.make_async_copy(xsp.at[ga[p]], xg[p], sg[p]),
                    pltpu.make_async_copy(wsp.at[wa[p]], wg[p], sg[p]))

        def gath_start(p):
            for d in gath_copies(p):
                d.start()

        def gath_wait(p):
            for d in gath_copies(p):
                d.wait()

        def scat_copy(u):
            return pltpu.make_async_copy(xg[u % 2], ssp.at[sa[u]], ss[u % 2])

        def mul(p):
            xgp = xg[p]
            wgp = wg[p]

            def body(v, _):
                for u in range(8):
                    r = rbase + (v * 8 + u) * 2
                    xv = plsc.load_gather(xgp, [r, cidx])
                    wv = plsc.load_gather(wgp, [r, cidx])
                    plsc.store_scatter(xgp, [r, cidx], xv * wv)
                return 0

            lax.fori_loop(0, W * BP // 16 // 8, body, 0)

        idx_start(0, 0)
        idx_wait(0, 0)
        gath_start(0)
        idx_start(1, 1)

        def lbody(q, _):
            j0 = 4 * q
            gath_wait(0)
            idx_wait(j0 + 1, 1)

            @pl.when(q > 0)
            def _():
                scat_copy(3).wait()

            gath_start(1)
            mul(0)
            scat_copy(0).start(add=True)
            idx_start(j0 + 2, 2)
            gath_wait(1)
            mul(1)
            scat_copy(1).start(add=True)
            scat_copy(0).wait()
            idx_wait(j0 + 2, 2)
            gath_start(0)
            idx_start(j0 + 3, 3)
            gath_wait(0)
            idx_wait(j0 + 3, 3)
            scat_copy(1).wait()
            gath_start(1)
            mul(0)
            scat_copy(2).start(add=True)

            @pl.when(q + 1 < Q)
            def _():
                idx_start(j0 + 4, 0)

            gath_wait(1)
            mul(1)
            scat_copy(3).start(add=True)
            scat_copy(2).wait()

            @pl.when(q + 1 < Q)
            def _():
                idx_wait(j0 + 4, 0)
                gath_start(0)
                idx_start(j0 + 5, 1)

            return 0

        lax.fori_loop(0, Q, lbody, 0)
        scat_copy(3).wait()

    run_pass(h_hbm, r_hbm, t_hbm)
    run_pass(t_hbm, ri_hbm, h_hbm)

    plsc.subcore_barrier()
    pltpu.sync_copy(ssp.at[pl.ds(r0, _SC_ROWS)],
                    out_hbm.at[c, pl.ds(r0, _SC_ROWS)])

  return _sc_propagate


def kernel(input_x, input_r, e2triple, triple2e, r2triple,
           emb_table, W_ih, W_hh, b_ih, b_hh, lin_W, lin_b):
    i32 = jnp.int32
    head = e2triple[0].astype(i32)
    rel_inv = (e2triple[2] % N_REL).astype(i32)
    tail = triple2e[1].astype(i32)
    rel = r2triple[0].astype(i32)

    pad_n = N_PAD - N_TRIPLES
    zpad = jnp.zeros((pad_n,), i32)
    rpad = jnp.full((pad_n,), N_REL, i32)
    head_p = jnp.concatenate([head, zpad]).reshape(2, 16, NW, W)
    tail_p = jnp.concatenate([tail, zpad]).reshape(2, 16, NW, W)
    rel_p = jnp.concatenate([rel, rpad]).reshape(2, 16, NW, W)
    ri_p = jnp.concatenate([rel_inv, rpad]).reshape(2, 16, NW, W)

    emb_seq = emb_table[input_r]
    emb_last = emb_table[N_REL][None]
    bsum = (b_ih + b_hh)[None]
    linb = lin_b[None]
    xb = jnp.tile(input_x.astype(i32), 16)[None]

    x0, att, wsm = _run_k1(emb_seq, emb_last, W_ih, W_hh, bsum, lin_W, linb, xb)

    wT = jnp.zeros((3, 32, BP), jnp.float32).at[:, :N_REL, :].set(
        jnp.transpose(wsm, (0, 2, 1)))
    zeros_pad = jnp.zeros((E_PAD, BP), jnp.float32)

    def coef_for(tnext):
        cpad = jnp.transpose(att[tnext, :, :tnext + 1], (1, 0))
        return jnp.tile(cpad, (1, 16))

    states2d = [x0]
    x_cur = x0
    for t in range(3):
        x2 = x_cur.reshape(E_PAD, BP)
        parts = _build_sc_propagate()(x2, wT[t], zeros_pad, head_p, tail_p,
                                      rel_p, ri_p)
        p2 = parts.reshape(2, ROWS_TC, 128)
        snew, xnext = _run_step(t, p2, coef_for(t + 1), states2d)
        states2d.append(snew)
        x_cur = xnext

    return jnp.transpose(x_cur.reshape(E_PAD, BP)[:E, :], (1, 0))
